# baseline (speedup 1.0000x reference)
import functools

import jax
import jax.numpy as jnp
from jax.experimental import pallas as pl
from jax.experimental.pallas import tpu as pltpu

_D_IN = 1024
_K = 64


_BM_E = 512
_BN_E = 2048


def _encode_body(x_ref, w_ref, b_ref, f_ref):
    acc = jnp.dot(x_ref[...], w_ref[...], preferred_element_type=jnp.float32)
    f_ref[...] = jnp.maximum(acc + b_ref[...], 0.0)


def _encode(x, W, b_enc):
    B, D = x.shape
    H = W.shape[1]
    grid = (H // _BN_E, B // _BM_E)
    return pl.pallas_call(
        _encode_body,
        grid=grid,
        in_specs=[
            pl.BlockSpec((_BM_E, D), lambda n, m: (m, 0)),
            pl.BlockSpec((D, _BN_E), lambda n, m: (0, n)),
            pl.BlockSpec((1, _BN_E), lambda n, m: (0, n)),
        ],
        out_specs=pl.BlockSpec((_BM_E, _BN_E), lambda n, m: (m, n)),
        out_shape=jax.ShapeDtypeStruct((B, H), jnp.float32),
    )(x, W, b_enc.reshape(1, H))


_BM_D = 512
_BH_D = 2048


def _decode_body(tau_ref, f_ref, w_ref, b_ref, ftopk_ref, recon_ref):
    h = pl.program_id(1)
    tau = tau_ref[0]
    f = f_ref[...]
    ft = jnp.where(f >= tau, f, 0.0)
    ftopk_ref[...] = ft
    contrib = jax.lax.dot_general(
        ft, w_ref[...], (((1,), (1,)), ((), ())),
        preferred_element_type=jnp.float32)

    @pl.when(h == 0)
    def _():
        recon_ref[...] = contrib + b_ref[...]

    @pl.when(h != 0)
    def _():
        recon_ref[...] += contrib


def _decode(tau, f, W, b_dec):
    B, H = f.shape
    D = W.shape[0]
    grid = (B // _BM_D, H // _BH_D)
    return pl.pallas_call(
        _decode_body,
        grid=grid,
        in_specs=[
            pl.BlockSpec(memory_space=pltpu.SMEM),
            pl.BlockSpec((_BM_D, _BH_D), lambda m, h: (m, h)),
            pl.BlockSpec((D, _BH_D), lambda m, h: (0, h)),
            pl.BlockSpec((1, D), lambda m, h: (0, 0)),
        ],
        out_specs=[
            pl.BlockSpec((_BM_D, _BH_D), lambda m, h: (m, h)),
            pl.BlockSpec((_BM_D, D), lambda m, h: (m, 0)),
        ],
        out_shape=[
            jax.ShapeDtypeStruct((B, H), jnp.float32),
            jax.ShapeDtypeStruct((B, D), jnp.float32),
        ],
    )(tau.reshape(1), f, W, b_dec.reshape(1, D))


def kernel(x, W, b_enc, b_dec):
    B = x.shape[0]
    numel = _K * B
    f = _encode(x, W, b_enc)
    tau = jnp.sort(f.reshape(-1))[f.size - numel]
    f_topk, recon = _decode(tau, f, W, b_dec)
    return (recon, f_topk)


# speedup vs baseline: 48.8023x; 48.8023x over previous
import functools

import jax
import jax.numpy as jnp
from jax import lax
from jax.experimental import pallas as pl
from jax.experimental.pallas import tpu as pltpu
from jax.experimental.pallas import tpu_sc as plsc

_NC = 2
_NS = 16
_NW = _NC * _NS
_LANES = 16

_NB1 = 32768
_NB2 = 65536
_CHUNK = 16384


_BM_E = 512
_BN_E = 2048


def _encode_body(x_ref, w_ref, b_ref, f_ref):
    acc = jnp.dot(x_ref[...], w_ref[...], preferred_element_type=jnp.float32)
    f_ref[...] = jnp.maximum(acc + b_ref[...], 0.0)


def _encode(x, W, b_enc):
    B, D = x.shape
    H = W.shape[1]
    grid = (H // _BN_E, B // _BM_E)
    return pl.pallas_call(
        _encode_body,
        grid=grid,
        in_specs=[
            pl.BlockSpec((_BM_E, D), lambda n, m: (m, 0)),
            pl.BlockSpec((D, _BN_E), lambda n, m: (0, n)),
            pl.BlockSpec((1, _BN_E), lambda n, m: (0, n)),
        ],
        out_specs=pl.BlockSpec((_BM_E, _BN_E), lambda n, m: (m, n)),
        out_shape=jax.ShapeDtypeStruct((B, H), jnp.float32),
    )(x, W, b_enc.reshape(1, H))


def _zero_vmem(ref, n):
    zeros = jnp.zeros((_LANES,), jnp.int32)

    def body(i, _):
        ref[pl.ds(i * _LANES, _LANES)] = zeros
        return 0

    lax.fori_loop(0, n // _LANES, body, 0)


def _hist_pass(f_hbm, out_hbm, buf0, buf1, hist, sem0, sem1, *,
               span, nbins, record):
    wid = lax.axis_index("s") * _NC + lax.axis_index("c")
    base = wid * span
    _zero_vmem(hist, nbins)

    ones = jnp.ones((_LANES,), jnp.int32)
    nvec = _CHUNK // _LANES

    def process(buf):
        def body(i, _):
            v = buf[pl.ds(i * _LANES, _LANES)]
            bits = lax.bitcast_convert_type(v, jnp.int32)
            record(hist, bits, ones)
            return 0

        lax.fori_loop(0, nvec, body, 0, unroll=8)

    nch = span // _CHUNK
    pltpu.async_copy(f_hbm.at[pl.ds(base, _CHUNK)], buf0, sem0)
    pltpu.async_copy(f_hbm.at[pl.ds(base + _CHUNK, _CHUNK)], buf1, sem1)

    def outer(t, _):
        off0 = base + (2 * t) * _CHUNK
        pltpu.make_async_copy(f_hbm.at[pl.ds(off0, _CHUNK)], buf0, sem0).wait()
        process(buf0)

        @pl.when(t < nch // 2 - 1)
        def _():
            pltpu.async_copy(
                f_hbm.at[pl.ds(off0 + 2 * _CHUNK, _CHUNK)], buf0, sem0)

        off1 = base + (2 * t + 1) * _CHUNK
        pltpu.make_async_copy(f_hbm.at[pl.ds(off1, _CHUNK)], buf1, sem1).wait()
        process(buf1)

        @pl.when(t < nch // 2 - 1)
        def _():
            pltpu.async_copy(
                f_hbm.at[pl.ds(off1 + 2 * _CHUNK, _CHUNK)], buf1, sem1)

        return 0

    lax.fori_loop(0, nch // 2, outer, 0)
    pltpu.sync_copy(hist, out_hbm.at[wid])


def _hist1(fflat):
    n = fflat.shape[0]
    span = n // _NW
    mesh = plsc.VectorSubcoreMesh(core_axis_name="c", subcore_axis_name="s")

    def record(hist, bits, ones):
        idx = lax.shift_right_logical(bits, 16)
        plsc.addupdate_scatter(hist, [idx], ones, mask=bits > 0)

    @functools.partial(
        pl.kernel, mesh=mesh,
        compiler_params=pltpu.CompilerParams(needs_layout_passes=False),
        out_type=jax.ShapeDtypeStruct((_NW, _NB1), jnp.int32),
        scratch_types=[
            pltpu.VMEM((_CHUNK,), jnp.float32),
            pltpu.VMEM((_CHUNK,), jnp.float32),
            pltpu.VMEM((_NB1,), jnp.int32),
            pltpu.SemaphoreType.DMA,
            pltpu.SemaphoreType.DMA,
        ],
    )
    def k(f_hbm, out_hbm, buf0, buf1, hist, sem0, sem1):
        _hist_pass(f_hbm, out_hbm, buf0, buf1, hist, sem0, sem1,
                   span=span, nbins=_NB1, record=record)

    return k(fflat)


def _hist2(fflat, b1_arr):
    n = fflat.shape[0]
    span = n // _NW
    mesh = plsc.VectorSubcoreMesh(core_axis_name="c", subcore_axis_name="s")

    @functools.partial(
        pl.kernel, mesh=mesh,
        compiler_params=pltpu.CompilerParams(needs_layout_passes=False),
        out_type=jax.ShapeDtypeStruct((_NW, _NB2), jnp.int32),
        scratch_types=[
            pltpu.VMEM((_CHUNK,), jnp.float32),
            pltpu.VMEM((_CHUNK,), jnp.float32),
            pltpu.VMEM((_NB2,), jnp.int32),
            pltpu.VMEM((_LANES,), jnp.int32),
            pltpu.SemaphoreType.DMA,
            pltpu.SemaphoreType.DMA,
        ],
    )
    def k(f_hbm, b1_hbm, out_hbm, buf0, buf1, hist, b1_buf, sem0, sem1):
        pltpu.sync_copy(b1_hbm.at[pl.ds(0, _LANES)], b1_buf)
        b1v = b1_buf[...]

        def record(hist_ref, bits, ones):
            top = lax.shift_right_logical(bits, 16)
            low = jnp.bitwise_and(bits, 0xFFFF)
            mask = jnp.logical_and(top == b1v, bits > 0)
            plsc.addupdate_scatter(hist_ref, [low], ones, mask=mask)

        _hist_pass(f_hbm, out_hbm, buf0, buf1, hist, sem0, sem1,
                   span=span, nbins=_NB2, record=record)

    return k(fflat, b1_arr)


def _suffix_sums(h):
    R = h.shape[0]
    kk = lax.broadcasted_iota(jnp.int32, (128, 128), 0)
    cc = lax.broadcasted_iota(jnp.int32, (128, 128), 1)
    lt = (kk >= cc).astype(jnp.float32)
    ws = jnp.dot(h, lt, preferred_element_type=jnp.float32,
                 precision=lax.Precision.HIGHEST)
    rr = lax.broadcasted_iota(jnp.int32, (R, R), 0)
    rc = lax.broadcasted_iota(jnp.int32, (R, R), 1)
    ut = (rc > rr).astype(jnp.float32)
    sre = jnp.dot(ut, ws[:, 0:1], preferred_element_type=jnp.float32,
                  precision=lax.Precision.HIGHEST)
    ss = ws + sre
    idxf = (lax.broadcasted_iota(jnp.int32, (R, 128), 0) * 128
            + lax.broadcasted_iota(jnp.int32, (R, 128), 1))
    return ss, idxf


def _scan1_body(numel, h_ref, b1_ref, need2_ref, tot_ref):
    h = jnp.sum(h_ref[...].astype(jnp.float32), axis=0)
    ss, idxf = _suffix_sums(h)
    mask = ss >= numel
    b1 = jnp.max(jnp.where(mask, idxf, -1))
    ssb1 = jnp.min(jnp.where(mask, ss, jnp.float32(3e38)))
    hb1 = jnp.max(jnp.where(idxf == b1, h, -1.0))
    need2 = numel - (ssb1 - hb1)
    total = jnp.max(ss)
    b1 = jnp.maximum(b1, 0)
    b1_ref[...] = jnp.full((1, 128), b1, jnp.int32)
    need2_ref[...] = jnp.full((1, 128), need2, jnp.float32)
    tot_ref[...] = jnp.full((1, 128), total, jnp.float32)


def _scan1(h1, numel):
    return pl.pallas_call(
        functools.partial(_scan1_body, float(numel)),
        in_specs=[pl.BlockSpec((_NW, _NB1 // 128, 128), lambda: (0, 0, 0))],
        out_specs=[
            pl.BlockSpec((1, 128), lambda: (0, 0)),
            pl.BlockSpec((1, 128), lambda: (0, 0)),
            pl.BlockSpec((1, 128), lambda: (0, 0)),
        ],
        out_shape=[
            jax.ShapeDtypeStruct((1, 128), jnp.int32),
            jax.ShapeDtypeStruct((1, 128), jnp.float32),
            jax.ShapeDtypeStruct((1, 128), jnp.float32),
        ],
    )(h1.reshape(_NW, _NB1 // 128, 128))


def _scan2_body(numel, h_ref, b1_ref, need2_ref, tot_ref, tau_ref):
    h = jnp.sum(h_ref[...].astype(jnp.float32), axis=0)
    ss, idxf = _suffix_sums(h)
    need2 = need2_ref[0, 0]
    mask = ss >= need2
    b2 = jnp.max(jnp.where(mask, idxf, -1))
    b2 = jnp.maximum(b2, 0)
    b1 = b1_ref[0, 0]
    tau_bits = jnp.bitwise_or(lax.shift_left(b1, 16), b2)
    tau = lax.bitcast_convert_type(tau_bits, jnp.float32)
    tau = jnp.where(tot_ref[0, 0] >= numel, tau, 0.0)
    tau_ref[...] = jnp.full((1, 128), tau, jnp.float32)


def _scan2(h2, b1_arr, need2_arr, tot_arr, numel):
    return pl.pallas_call(
        functools.partial(_scan2_body, float(numel)),
        in_specs=[
            pl.BlockSpec((_NW, _NB2 // 128, 128), lambda: (0, 0, 0)),
            pl.BlockSpec((1, 128), lambda: (0, 0)),
            pl.BlockSpec((1, 128), lambda: (0, 0)),
            pl.BlockSpec((1, 128), lambda: (0, 0)),
        ],
        out_specs=pl.BlockSpec((1, 128), lambda: (0, 0)),
        out_shape=jax.ShapeDtypeStruct((1, 128), jnp.float32),
    )(h2.reshape(_NW, _NB2 // 128, 128), b1_arr, need2_arr, tot_arr)


_BM_D = 512
_BH_D = 2048


def _decode_body(tau_ref, f_ref, w_ref, b_ref, ftopk_ref, recon_ref):
    h = pl.program_id(1)
    tau = tau_ref[0, 0]
    f = f_ref[...]
    ft = jnp.where(f >= tau, f, 0.0)
    ftopk_ref[...] = ft
    contrib = lax.dot_general(
        ft, w_ref[...], (((1,), (1,)), ((), ())),
        preferred_element_type=jnp.float32)

    @pl.when(h == 0)
    def _():
        recon_ref[...] = contrib + b_ref[...]

    @pl.when(h != 0)
    def _():
        recon_ref[...] += contrib


def _decode(tau_arr, f, W, b_dec):
    B, H = f.shape
    D = W.shape[0]
    grid = (B // _BM_D, H // _BH_D)
    return pl.pallas_call(
        _decode_body,
        grid=grid,
        in_specs=[
            pl.BlockSpec(memory_space=pltpu.SMEM),
            pl.BlockSpec((_BM_D, _BH_D), lambda m, h: (m, h)),
            pl.BlockSpec((D, _BH_D), lambda m, h: (0, h)),
            pl.BlockSpec((1, D), lambda m, h: (0, 0)),
        ],
        out_specs=[
            pl.BlockSpec((_BM_D, _BH_D), lambda m, h: (m, h)),
            pl.BlockSpec((_BM_D, D), lambda m, h: (m, 0)),
        ],
        out_shape=[
            jax.ShapeDtypeStruct((B, H), jnp.float32),
            jax.ShapeDtypeStruct((B, D), jnp.float32),
        ],
    )(tau_arr, f, W, b_dec.reshape(1, D))


def kernel(x, W, b_enc, b_dec):
    B = x.shape[0]
    numel = 64 * B
    f = _encode(x, W, b_enc)
    fflat = f.reshape(-1)
    h1 = _hist1(fflat)
    b1_arr, need2_arr, tot_arr = _scan1(h1, numel)
    h2 = _hist2(fflat, b1_arr.reshape(128))
    tau_arr = _scan2(h2, b1_arr, need2_arr, tot_arr, numel)
    f_topk, recon = _decode(tau_arr, f, W, b_dec)
    return (recon, f_topk)


# speedup vs baseline: 110.4141x; 2.2625x over previous
import functools

import jax
import jax.numpy as jnp
from jax import lax
from jax.experimental import pallas as pl
from jax.experimental.pallas import tpu as pltpu
from jax.experimental.pallas import tpu_sc as plsc

_NC = 2
_NS = 16
_NW = _NC * _NS
_LANES = 16

_NB1 = 32768
_NB2 = 65536
_CHUNK1 = 32768
_CHUNK2 = 16384


_BM_E = 512
_BN_E = 2048


def _encode_body(x_ref, w_ref, b_ref, f_ref):
    acc = jnp.dot(x_ref[...], w_ref[...], preferred_element_type=jnp.float32)
    f_ref[...] = jnp.maximum(acc + b_ref[...], 0.0)


def _encode(x, W, b_enc):
    B, D = x.shape
    H = W.shape[1]
    grid = (H // _BN_E, B // _BM_E)
    return pl.pallas_call(
        _encode_body,
        grid=grid,
        in_specs=[
            pl.BlockSpec((_BM_E, D), lambda n, m: (m, 0)),
            pl.BlockSpec((D, _BN_E), lambda n, m: (0, n)),
            pl.BlockSpec((1, _BN_E), lambda n, m: (0, n)),
        ],
        out_specs=pl.BlockSpec((_BM_E, _BN_E), lambda n, m: (m, n)),
        out_shape=jax.ShapeDtypeStruct((B, H), jnp.float32),
    )(x, W, b_enc.reshape(1, H))


def _zero_vmem(ref, n):
    zeros = jnp.zeros((_LANES,), jnp.int32)

    def body(i, _):
        ref[pl.ds(i * _LANES, _LANES)] = zeros
        return 0

    lax.fori_loop(0, n // _LANES, body, 0)


def _hist_pass(f_hbm, out_hbm, buf0, buf1, hist, sem0, sem1, *,
               span, nbins, chunk, record):
    wid = lax.axis_index("s") * _NC + lax.axis_index("c")
    base = wid * span
    _zero_vmem(hist, nbins)

    ones = jnp.ones((_LANES,), jnp.int32)
    nvec = chunk // _LANES

    def process(buf):
        @plsc.parallel_loop(0, nvec, unroll=8)
        def _(i):
            v = buf[pl.ds(i * _LANES, _LANES)]
            bits = lax.bitcast_convert_type(v, jnp.int32)
            record(hist, bits, ones)

    nch = span // chunk
    pltpu.async_copy(f_hbm.at[pl.ds(base, chunk)], buf0, sem0)
    pltpu.async_copy(f_hbm.at[pl.ds(base + chunk, chunk)], buf1, sem1)

    def outer(t, _):
        off0 = base + (2 * t) * chunk
        pltpu.make_async_copy(f_hbm.at[pl.ds(off0, chunk)], buf0, sem0).wait()
        process(buf0)

        @pl.when(t < nch // 2 - 1)
        def _():
            pltpu.async_copy(
                f_hbm.at[pl.ds(off0 + 2 * chunk, chunk)], buf0, sem0)

        off1 = base + (2 * t + 1) * chunk
        pltpu.make_async_copy(f_hbm.at[pl.ds(off1, chunk)], buf1, sem1).wait()
        process(buf1)

        @pl.when(t < nch // 2 - 1)
        def _():
            pltpu.async_copy(
                f_hbm.at[pl.ds(off1 + 2 * chunk, chunk)], buf1, sem1)

        return 0

    lax.fori_loop(0, nch // 2, outer, 0)
    pltpu.sync_copy(hist, out_hbm.at[wid])


def _hist1(fflat):
    n = fflat.shape[0]
    span = n // _NW
    mesh = plsc.VectorSubcoreMesh(core_axis_name="c", subcore_axis_name="s")

    def record(hist, bits, ones):
        idx = lax.shift_right_logical(bits, 16)
        plsc.addupdate_scatter(hist, [idx], ones, mask=bits > 0)

    @functools.partial(
        pl.kernel, mesh=mesh,
        compiler_params=pltpu.CompilerParams(needs_layout_passes=False),
        out_type=jax.ShapeDtypeStruct((_NW, _NB1), jnp.int32),
        scratch_types=[
            pltpu.VMEM((_CHUNK1,), jnp.float32),
            pltpu.VMEM((_CHUNK1,), jnp.float32),
            pltpu.VMEM((_NB1,), jnp.int32),
            pltpu.SemaphoreType.DMA,
            pltpu.SemaphoreType.DMA,
        ],
    )
    def k(f_hbm, out_hbm, buf0, buf1, hist, sem0, sem1):
        _hist_pass(f_hbm, out_hbm, buf0, buf1, hist, sem0, sem1,
                   span=span, nbins=_NB1, chunk=_CHUNK1, record=record)

    return k(fflat)


def _hist2(fflat, b1_arr):
    n = fflat.shape[0]
    span = n // _NW
    mesh = plsc.VectorSubcoreMesh(core_axis_name="c", subcore_axis_name="s")

    @functools.partial(
        pl.kernel, mesh=mesh,
        compiler_params=pltpu.CompilerParams(needs_layout_passes=False),
        out_type=jax.ShapeDtypeStruct((_NW, _NB2), jnp.int32),
        scratch_types=[
            pltpu.VMEM((_CHUNK2,), jnp.float32),
            pltpu.VMEM((_CHUNK2,), jnp.float32),
            pltpu.VMEM((_NB2,), jnp.int32),
            pltpu.VMEM((_LANES,), jnp.int32),
            pltpu.SemaphoreType.DMA,
            pltpu.SemaphoreType.DMA,
        ],
    )
    def k(f_hbm, b1_hbm, out_hbm, buf0, buf1, hist, b1_buf, sem0, sem1):
        pltpu.sync_copy(b1_hbm.at[pl.ds(0, _LANES)], b1_buf)
        b1v = b1_buf[...]

        def record(hist_ref, bits, ones):
            top = lax.shift_right_logical(bits, 16)
            low = jnp.bitwise_and(bits, 0xFFFF)
            mask = jnp.logical_and(top == b1v, bits > 0)
            plsc.addupdate_scatter(hist_ref, [low], ones, mask=mask)

        _hist_pass(f_hbm, out_hbm, buf0, buf1, hist, sem0, sem1,
                   span=span, nbins=_NB2, chunk=_CHUNK2, record=record)

    return k(fflat, b1_arr)


def _suffix_sums(h):
    R = h.shape[0]
    kk = lax.broadcasted_iota(jnp.int32, (128, 128), 0)
    cc = lax.broadcasted_iota(jnp.int32, (128, 128), 1)
    lt = (kk >= cc).astype(jnp.float32)
    ws = jnp.dot(h, lt, preferred_element_type=jnp.float32,
                 precision=lax.Precision.HIGHEST)
    rr = lax.broadcasted_iota(jnp.int32, (R, R), 0)
    rc = lax.broadcasted_iota(jnp.int32, (R, R), 1)
    ut = (rc > rr).astype(jnp.float32)
    sre = jnp.dot(ut, ws[:, 0:1], preferred_element_type=jnp.float32,
                  precision=lax.Precision.HIGHEST)
    ss = ws + sre
    idxf = (lax.broadcasted_iota(jnp.int32, (R, 128), 0) * 128
            + lax.broadcasted_iota(jnp.int32, (R, 128), 1))
    return ss, idxf


def _scan1_body(numel, h_ref, b1_ref, need2_ref, tot_ref):
    h = jnp.sum(h_ref[...].astype(jnp.float32), axis=0)
    ss, idxf = _suffix_sums(h)
    mask = ss >= numel
    b1 = jnp.max(jnp.where(mask, idxf, -1))
    ssb1 = jnp.min(jnp.where(mask, ss, jnp.float32(3e38)))
    hb1 = jnp.max(jnp.where(idxf == b1, h, -1.0))
    need2 = numel - (ssb1 - hb1)
    total = jnp.max(ss)
    b1 = jnp.maximum(b1, 0)
    b1_ref[...] = jnp.full((1, 128), b1, jnp.int32)
    need2_ref[...] = jnp.full((1, 128), need2, jnp.float32)
    tot_ref[...] = jnp.full((1, 128), total, jnp.float32)


def _scan1(h1, numel):
    return pl.pallas_call(
        functools.partial(_scan1_body, float(numel)),
        in_specs=[pl.BlockSpec((_NW, _NB1 // 128, 128), lambda: (0, 0, 0))],
        out_specs=[
            pl.BlockSpec((1, 128), lambda: (0, 0)),
            pl.BlockSpec((1, 128), lambda: (0, 0)),
            pl.BlockSpec((1, 128), lambda: (0, 0)),
        ],
        out_shape=[
            jax.ShapeDtypeStruct((1, 128), jnp.int32),
            jax.ShapeDtypeStruct((1, 128), jnp.float32),
            jax.ShapeDtypeStruct((1, 128), jnp.float32),
        ],
    )(h1.reshape(_NW, _NB1 // 128, 128))


def _scan2_body(numel, h_ref, b1_ref, need2_ref, tot_ref, tau_ref):
    h = jnp.sum(h_ref[...].astype(jnp.float32), axis=0)
    ss, idxf = _suffix_sums(h)
    need2 = need2_ref[0, 0]
    mask = ss >= need2
    b2 = jnp.max(jnp.where(mask, idxf, -1))
    b2 = jnp.maximum(b2, 0)
    b1 = b1_ref[0, 0]
    tau_bits = jnp.bitwise_or(lax.shift_left(b1, 16), b2)
    tau = lax.bitcast_convert_type(tau_bits, jnp.float32)
    tau = jnp.where(tot_ref[0, 0] >= numel, tau, 0.0)
    tau_ref[...] = jnp.full((1, 128), tau, jnp.float32)


def _scan2(h2, b1_arr, need2_arr, tot_arr, numel):
    return pl.pallas_call(
        functools.partial(_scan2_body, float(numel)),
        in_specs=[
            pl.BlockSpec((_NW, _NB2 // 128, 128), lambda: (0, 0, 0)),
            pl.BlockSpec((1, 128), lambda: (0, 0)),
            pl.BlockSpec((1, 128), lambda: (0, 0)),
            pl.BlockSpec((1, 128), lambda: (0, 0)),
        ],
        out_specs=pl.BlockSpec((1, 128), lambda: (0, 0)),
        out_shape=jax.ShapeDtypeStruct((1, 128), jnp.float32),
    )(h2.reshape(_NW, _NB2 // 128, 128), b1_arr, need2_arr, tot_arr)


_BM_D = 512
_BH_D = 2048


def _decode_body(tau_ref, f_ref, w_ref, b_ref, ftopk_ref, recon_ref):
    h = pl.program_id(1)
    tau = tau_ref[0, 0]
    f = f_ref[...]
    ft = jnp.where(f >= tau, f, 0.0)
    ftopk_ref[...] = ft
    contrib = lax.dot_general(
        ft, w_ref[...], (((1,), (1,)), ((), ())),
        preferred_element_type=jnp.float32)

    @pl.when(h == 0)
    def _():
        recon_ref[...] = contrib + b_ref[...]

    @pl.when(h != 0)
    def _():
        recon_ref[...] += contrib


def _decode(tau_arr, f, W, b_dec):
    B, H = f.shape
    D = W.shape[0]
    grid = (B // _BM_D, H // _BH_D)
    return pl.pallas_call(
        _decode_body,
        grid=grid,
        in_specs=[
            pl.BlockSpec(memory_space=pltpu.SMEM),
            pl.BlockSpec((_BM_D, _BH_D), lambda m, h: (m, h)),
            pl.BlockSpec((D, _BH_D), lambda m, h: (0, h)),
            pl.BlockSpec((1, D), lambda m, h: (0, 0)),
        ],
        out_specs=[
            pl.BlockSpec((_BM_D, _BH_D), lambda m, h: (m, h)),
            pl.BlockSpec((_BM_D, D), lambda m, h: (m, 0)),
        ],
        out_shape=[
            jax.ShapeDtypeStruct((B, H), jnp.float32),
            jax.ShapeDtypeStruct((B, D), jnp.float32),
        ],
    )(tau_arr, f, W, b_dec.reshape(1, D))


def kernel(x, W, b_enc, b_dec):
    B = x.shape[0]
    numel = 64 * B
    f = _encode(x, W, b_enc)
    fflat = f.reshape(-1)
    h1 = _hist1(fflat)
    b1_arr, need2_arr, tot_arr = _scan1(h1, numel)
    h2 = _hist2(fflat, b1_arr.reshape(128))
    tau_arr = _scan2(h2, b1_arr, need2_arr, tot_arr, numel)
    f_topk, recon = _decode(tau_arr, f, W, b_dec)
    return (recon, f_topk)


# speedup vs baseline: 148.5996x; 1.3458x over previous
import functools

import jax
import jax.numpy as jnp
from jax import lax
from jax.experimental import pallas as pl
from jax.experimental.pallas import tpu as pltpu
from jax.experimental.pallas import tpu_sc as plsc

_NC = 2
_NS = 16
_NW = _NC * _NS
_LANES = 16

_NB1 = 32768
_NB2 = 32768
_CH_ROWS = 8
_CH_COLS = 2048


_BM_E = 512
_BN_E = 2048


def _encode_body(x_ref, w_ref, b_ref, f_ref):
    acc = jnp.dot(x_ref[...], w_ref[...], preferred_element_type=jnp.float32)
    f_ref[...] = jnp.maximum(acc + b_ref[...], 0.0)


def _encode(x, W, b_enc):
    B, D = x.shape
    H = W.shape[1]
    grid = (H // _BN_E, B // _BM_E)
    return pl.pallas_call(
        _encode_body,
        grid=grid,
        in_specs=[
            pl.BlockSpec((_BM_E, D), lambda n, m: (m, 0)),
            pl.BlockSpec((D, _BN_E), lambda n, m: (0, n)),
            pl.BlockSpec((1, _BN_E), lambda n, m: (0, n)),
        ],
        out_specs=pl.BlockSpec((_BM_E, _BN_E), lambda n, m: (m, n)),
        out_shape=jax.ShapeDtypeStruct((B, H), jnp.float32),
    )(x, W, b_enc.reshape(1, H))


def _zero_vmem(ref, n):
    zeros = jnp.zeros((_LANES,), jnp.int32)

    def body(i, _):
        ref[pl.ds(i * _LANES, _LANES)] = zeros
        return 0

    lax.fori_loop(0, n // _LANES, body, 0)


def _hist_pass(f_hbm, out_hbm, buf0, buf1, hist, sem0, sem1, *,
               nbins, record):
    wid = lax.axis_index("s") * _NC + lax.axis_index("c")
    nrows, ncols = f_hbm.shape
    rows_per_w = nrows // _NW
    row0 = wid * rows_per_w
    ncolc = ncols // _CH_COLS
    nch = (rows_per_w // _CH_ROWS) * ncolc
    _zero_vmem(hist, nbins)

    ones = jnp.ones((_LANES,), jnp.int32)
    nvec = _CH_COLS // _LANES

    def _slc(t):
        return (pl.ds(row0 + (t // ncolc) * _CH_ROWS, _CH_ROWS),
                pl.ds((t % ncolc) * _CH_COLS, _CH_COLS))

    def start(t, buf, sem):
        r, c = _slc(t)
        pltpu.async_copy(f_hbm.at[r, c], buf, sem)

    def wait(t, buf, sem):
        r, c = _slc(t)
        pltpu.make_async_copy(f_hbm.at[r, c], buf, sem).wait()

    def process(buf):
        for r in range(_CH_ROWS):
            @plsc.parallel_loop(0, nvec, unroll=8)
            def _(i):
                v = buf[r, pl.ds(i * _LANES, _LANES)]
                bits = lax.bitcast_convert_type(v, jnp.int32)
                record(hist, bits, ones)

    start(0, buf0, sem0)
    start(1, buf1, sem1)

    def outer(t, _):
        wait(2 * t, buf0, sem0)
        process(buf0)

        @pl.when(t < nch // 2 - 1)
        def _():
            start(2 * t + 2, buf0, sem0)

        wait(2 * t + 1, buf1, sem1)
        process(buf1)

        @pl.when(t < nch // 2 - 1)
        def _():
            start(2 * t + 3, buf1, sem1)

        return 0

    lax.fori_loop(0, nch // 2, outer, 0)
    pltpu.sync_copy(hist, out_hbm.at[wid])


def _hist1(f):
    mesh = plsc.VectorSubcoreMesh(core_axis_name="c", subcore_axis_name="s")

    def record(hist, bits, ones):
        idx = lax.shift_right_logical(bits, 16)
        plsc.addupdate_scatter(hist, [idx], ones, mask=bits > 0)

    @functools.partial(
        pl.kernel, mesh=mesh,
        compiler_params=pltpu.CompilerParams(needs_layout_passes=False),
        out_type=jax.ShapeDtypeStruct((_NW, _NB1), jnp.int32),
        scratch_types=[
            pltpu.VMEM((_CH_ROWS, _CH_COLS), jnp.float32),
            pltpu.VMEM((_CH_ROWS, _CH_COLS), jnp.float32),
            pltpu.VMEM((_NB1,), jnp.int32),
            pltpu.SemaphoreType.DMA,
            pltpu.SemaphoreType.DMA,
        ],
    )
    def k(f_hbm, out_hbm, buf0, buf1, hist, sem0, sem1):
        _hist_pass(f_hbm, out_hbm, buf0, buf1, hist, sem0, sem1,
                   nbins=_NB1, record=record)

    return k(f)


def _hist2(f, b1_arr):
    mesh = plsc.VectorSubcoreMesh(core_axis_name="c", subcore_axis_name="s")

    @functools.partial(
        pl.kernel, mesh=mesh,
        compiler_params=pltpu.CompilerParams(needs_layout_passes=False),
        out_type=jax.ShapeDtypeStruct((_NW, _NB2), jnp.int32),
        scratch_types=[
            pltpu.VMEM((_CH_ROWS, _CH_COLS), jnp.float32),
            pltpu.VMEM((_CH_ROWS, _CH_COLS), jnp.float32),
            pltpu.VMEM((_NB2,), jnp.int32),
            pltpu.VMEM((_LANES,), jnp.int32),
            pltpu.SemaphoreType.DMA,
            pltpu.SemaphoreType.DMA,
        ],
    )
    def k(f_hbm, b1_hbm, out_hbm, buf0, buf1, hist, b1_buf, sem0, sem1):
        pltpu.sync_copy(b1_hbm.at[pl.ds(0, _LANES)], b1_buf)
        b1v = b1_buf[...]

        def record(hist_ref, bits, ones):
            top = lax.shift_right_logical(bits, 16)
            q = jnp.bitwise_and(lax.shift_right_logical(bits, 1), 0x7FFF)
            mask = jnp.logical_and(top == b1v, bits > 0)
            plsc.addupdate_scatter(hist_ref, [q], ones, mask=mask)

        _hist_pass(f_hbm, out_hbm, buf0, buf1, hist, sem0, sem1,
                   nbins=_NB2, record=record)

    return k(f, b1_arr)


def _suffix_sums(h):
    R = h.shape[0]
    kk = lax.broadcasted_iota(jnp.int32, (128, 128), 0)
    cc = lax.broadcasted_iota(jnp.int32, (128, 128), 1)
    lt = (kk >= cc).astype(jnp.float32)
    ws = jnp.dot(h, lt, preferred_element_type=jnp.float32,
                 precision=lax.Precision.HIGHEST)
    rr = lax.broadcasted_iota(jnp.int32, (R, R), 0)
    rc = lax.broadcasted_iota(jnp.int32, (R, R), 1)
    ut = (rc > rr).astype(jnp.float32)
    sre = jnp.dot(ut, ws[:, 0:1], preferred_element_type=jnp.float32,
                  precision=lax.Precision.HIGHEST)
    ss = ws + sre
    idxf = (lax.broadcasted_iota(jnp.int32, (R, 128), 0) * 128
            + lax.broadcasted_iota(jnp.int32, (R, 128), 1))
    return ss, idxf


def _scan1_body(numel, h_ref, b1_ref, need2_ref, tot_ref):
    h = jnp.sum(h_ref[...].astype(jnp.float32), axis=0)
    ss, idxf = _suffix_sums(h)
    mask = ss >= numel
    b1 = jnp.max(jnp.where(mask, idxf, -1))
    ssb1 = jnp.min(jnp.where(mask, ss, jnp.float32(3e38)))
    hb1 = jnp.max(jnp.where(idxf == b1, h, -1.0))
    need2 = numel - (ssb1 - hb1)
    total = jnp.max(ss)
    b1 = jnp.maximum(b1, 0)
    b1_ref[...] = jnp.full((1, 128), b1, jnp.int32)
    need2_ref[...] = jnp.full((1, 128), need2, jnp.float32)
    tot_ref[...] = jnp.full((1, 128), total, jnp.float32)


def _scan1(h1, numel):
    return pl.pallas_call(
        functools.partial(_scan1_body, float(numel)),
        in_specs=[pl.BlockSpec((_NW, _NB1 // 128, 128), lambda: (0, 0, 0))],
        out_specs=[
            pl.BlockSpec((1, 128), lambda: (0, 0)),
            pl.BlockSpec((1, 128), lambda: (0, 0)),
            pl.BlockSpec((1, 128), lambda: (0, 0)),
        ],
        out_shape=[
            jax.ShapeDtypeStruct((1, 128), jnp.int32),
            jax.ShapeDtypeStruct((1, 128), jnp.float32),
            jax.ShapeDtypeStruct((1, 128), jnp.float32),
        ],
    )(h1.reshape(_NW, _NB1 // 128, 128))


def _tau_from_hists(numel, h_ref, b1_ref, need2_ref, tot_ref):
    h = jnp.sum(h_ref[...].astype(jnp.float32), axis=0)
    ss, idxf = _suffix_sums(h)
    need2 = need2_ref[0, 0]
    mask = ss >= need2
    b2 = jnp.max(jnp.where(mask, idxf, -1))
    b2 = jnp.maximum(b2, 0)
    b1 = b1_ref[0, 0]
    tau_bits = jnp.bitwise_or(lax.shift_left(b1, 16), lax.shift_left(b2, 1))
    tau = lax.bitcast_convert_type(tau_bits, jnp.float32)
    return jnp.where(tot_ref[0, 0] >= numel, tau, 0.0)


_BM_D = 512
_BH_D = 2048


def _decode_body(numel, h2_ref, b1_ref, need2_ref, tot_ref, f_ref, w_ref,
                 b_ref, ftopk_ref, recon_ref, tau_s):
    m = pl.program_id(0)
    h = pl.program_id(1)

    @pl.when(jnp.logical_and(m == 0, h == 0))
    def _():
        tau_s[0] = _tau_from_hists(numel, h2_ref, b1_ref, need2_ref, tot_ref)

    tau = tau_s[0]
    f = f_ref[...]
    ft = jnp.where(f >= tau, f, 0.0)
    ftopk_ref[...] = ft
    contrib = lax.dot_general(
        ft, w_ref[...], (((1,), (1,)), ((), ())),
        preferred_element_type=jnp.float32)

    @pl.when(h == 0)
    def _():
        recon_ref[...] = contrib + b_ref[...]

    @pl.when(h != 0)
    def _():
        recon_ref[...] += contrib


def _decode(h2, b1_arr, need2_arr, tot_arr, f, W, b_dec, numel):
    B, H = f.shape
    D = W.shape[0]
    grid = (B // _BM_D, H // _BH_D)
    return pl.pallas_call(
        functools.partial(_decode_body, float(numel)),
        grid=grid,
        in_specs=[
            pl.BlockSpec((_NW, _NB2 // 128, 128), lambda m, h: (0, 0, 0)),
            pl.BlockSpec(memory_space=pltpu.SMEM),
            pl.BlockSpec(memory_space=pltpu.SMEM),
            pl.BlockSpec(memory_space=pltpu.SMEM),
            pl.BlockSpec((_BM_D, _BH_D), lambda m, h: (m, h)),
            pl.BlockSpec((D, _BH_D), lambda m, h: (0, h)),
            pl.BlockSpec((1, D), lambda m, h: (0, 0)),
        ],
        out_specs=[
            pl.BlockSpec((_BM_D, _BH_D), lambda m, h: (m, h)),
            pl.BlockSpec((_BM_D, D), lambda m, h: (m, 0)),
        ],
        out_shape=[
            jax.ShapeDtypeStruct((B, H), jnp.float32),
            jax.ShapeDtypeStruct((B, D), jnp.float32),
        ],
        scratch_shapes=[pltpu.SMEM((1,), jnp.float32)],
    )(h2.reshape(_NW, _NB2 // 128, 128), b1_arr, need2_arr, tot_arr,
      f, W, b_dec.reshape(1, D))


def kernel(x, W, b_enc, b_dec):
    B = x.shape[0]
    numel = 64 * B
    f = _encode(x, W, b_enc)
    h1 = _hist1(f)
    b1_arr, need2_arr, tot_arr = _scan1(h1, numel)
    h2 = _hist2(f, b1_arr.reshape(128))
    f_topk, recon = _decode(h2, b1_arr, need2_arr, tot_arr, f, W, b_dec,
                            numel)
    return (recon, f_topk)


# speedup vs baseline: 160.3124x; 1.0788x over previous
import functools

import jax
import jax.numpy as jnp
from jax import lax
from jax.experimental import pallas as pl
from jax.experimental.pallas import tpu as pltpu
from jax.experimental.pallas import tpu_sc as plsc

_NC = 2
_NS = 16
_NW = _NC * _NS
_LANES = 16

_NB1 = 32768
_NB2 = 32768
_CH_ROWS = 8
_CH_COLS = 2048


_BM_E = 512
_BN_E = 2048


def _encode_body(x_ref, w_ref, b_ref, f_ref):
    acc = jnp.dot(x_ref[...], w_ref[...], preferred_element_type=jnp.float32)
    f_ref[...] = jnp.maximum(acc + b_ref[...], 0.0)


def _encode(x, W, b_enc):
    B, D = x.shape
    H = W.shape[1]
    grid = (H // _BN_E, B // _BM_E)
    return pl.pallas_call(
        _encode_body,
        grid=grid,
        in_specs=[
            pl.BlockSpec((_BM_E, D), lambda n, m: (m, 0)),
            pl.BlockSpec((D, _BN_E), lambda n, m: (0, n)),
            pl.BlockSpec((1, _BN_E), lambda n, m: (0, n)),
        ],
        out_specs=pl.BlockSpec((_BM_E, _BN_E), lambda n, m: (m, n)),
        out_shape=jax.ShapeDtypeStruct((B, H), jnp.float32),
    )(x, W, b_enc.reshape(1, H))


def _zero_vmem(ref, n):
    zeros = jnp.zeros((_LANES,), jnp.int32)

    def body(i, _):
        ref[pl.ds(i * _LANES, _LANES)] = zeros
        return 0

    lax.fori_loop(0, n // _LANES, body, 0)


def _hist_pass(f_hbm, out_hbm, buf0, buf1, hist, sem0, sem1, *,
               nbins, record):
    wid = lax.axis_index("s") * _NC + lax.axis_index("c")
    nrows, ncols = f_hbm.shape
    rows_per_w = nrows // _NW
    row0 = wid * rows_per_w
    ncolc = ncols // _CH_COLS
    nch = (rows_per_w // _CH_ROWS) * ncolc
    _zero_vmem(hist, nbins)

    ones = jnp.ones((_LANES,), jnp.int32)
    nvec = _CH_COLS // _LANES

    def _slc(t):
        return (pl.ds(row0 + (t // ncolc) * _CH_ROWS, _CH_ROWS),
                pl.ds((t % ncolc) * _CH_COLS, _CH_COLS))

    def start(t, buf, sem):
        r, c = _slc(t)
        pltpu.async_copy(f_hbm.at[r, c], buf, sem)

    def wait(t, buf, sem):
        r, c = _slc(t)
        pltpu.make_async_copy(f_hbm.at[r, c], buf, sem).wait()

    def process(buf):
        for r in range(_CH_ROWS):
            @plsc.parallel_loop(0, nvec, unroll=8)
            def _(i):
                v = buf[r, pl.ds(i * _LANES, _LANES)]
                bits = lax.bitcast_convert_type(v, jnp.int32)
                record(hist, bits, ones)

    start(0, buf0, sem0)
    start(1, buf1, sem1)

    def outer(t, _):
        wait(2 * t, buf0, sem0)
        process(buf0)

        @pl.when(t < nch // 2 - 1)
        def _():
            start(2 * t + 2, buf0, sem0)

        wait(2 * t + 1, buf1, sem1)
        process(buf1)

        @pl.when(t < nch // 2 - 1)
        def _():
            start(2 * t + 3, buf1, sem1)

        return 0

    lax.fori_loop(0, nch // 2, outer, 0)
    pltpu.sync_copy(hist, out_hbm.at[wid])


def _hist1(f):
    mesh = plsc.VectorSubcoreMesh(core_axis_name="c", subcore_axis_name="s")

    def record(hist, bits, ones):
        idx = lax.shift_right_logical(bits, 16)
        plsc.addupdate_scatter(hist, [idx], ones, mask=bits > 0)

    @functools.partial(
        pl.kernel, mesh=mesh,
        compiler_params=pltpu.CompilerParams(needs_layout_passes=False),
        out_type=jax.ShapeDtypeStruct((_NW, _NB1), jnp.int32),
        scratch_types=[
            pltpu.VMEM((_CH_ROWS, _CH_COLS), jnp.float32),
            pltpu.VMEM((_CH_ROWS, _CH_COLS), jnp.float32),
            pltpu.VMEM((_NB1,), jnp.int32),
            pltpu.SemaphoreType.DMA,
            pltpu.SemaphoreType.DMA,
        ],
    )
    def k(f_hbm, out_hbm, buf0, buf1, hist, sem0, sem1):
        _hist_pass(f_hbm, out_hbm, buf0, buf1, hist, sem0, sem1,
                   nbins=_NB1, record=record)

    return k(f)


def _hist2(f, b1_arr):
    mesh = plsc.VectorSubcoreMesh(core_axis_name="c", subcore_axis_name="s")

    @functools.partial(
        pl.kernel, mesh=mesh,
        compiler_params=pltpu.CompilerParams(needs_layout_passes=False),
        out_type=jax.ShapeDtypeStruct((_NW, _NB2), jnp.int32),
        scratch_types=[
            pltpu.VMEM((_CH_ROWS, _CH_COLS), jnp.float32),
            pltpu.VMEM((_CH_ROWS, _CH_COLS), jnp.float32),
            pltpu.VMEM((_NB2,), jnp.int32),
            pltpu.VMEM((_LANES,), jnp.int32),
            pltpu.SemaphoreType.DMA,
            pltpu.SemaphoreType.DMA,
        ],
    )
    def k(f_hbm, b1_hbm, out_hbm, buf0, buf1, hist, b1_buf, sem0, sem1):
        pltpu.sync_copy(b1_hbm.at[pl.ds(0, _LANES)], b1_buf)
        b1v = b1_buf[...]

        def record(hist_ref, bits, ones):
            top = lax.shift_right_logical(bits, 16)
            q = jnp.bitwise_and(lax.shift_right_logical(bits, 1), 0x7FFF)
            mask = jnp.logical_and(top == b1v, bits > 0)
            plsc.addupdate_scatter(hist_ref, [q], ones, mask=mask)

        _hist_pass(f_hbm, out_hbm, buf0, buf1, hist, sem0, sem1,
                   nbins=_NB2, record=record)

    return k(f, b1_arr)


def _suffix_sums(h):
    R = h.shape[0]
    kk = lax.broadcasted_iota(jnp.int32, (128, 128), 0)
    cc = lax.broadcasted_iota(jnp.int32, (128, 128), 1)
    lt = (kk >= cc).astype(jnp.float32)
    ws = jnp.dot(h, lt, preferred_element_type=jnp.float32,
                 precision=lax.Precision.HIGHEST)
    rr = lax.broadcasted_iota(jnp.int32, (R, R), 0)
    rc = lax.broadcasted_iota(jnp.int32, (R, R), 1)
    ut = (rc > rr).astype(jnp.float32)
    sre = jnp.dot(ut, ws[:, 0:1], preferred_element_type=jnp.float32,
                  precision=lax.Precision.HIGHEST)
    ss = ws + sre
    idxf = (lax.broadcasted_iota(jnp.int32, (R, 128), 0) * 128
            + lax.broadcasted_iota(jnp.int32, (R, 128), 1))
    return ss, idxf


def _scan1_body(numel, h_ref, b1_ref, need2_ref, tot_ref):
    h = jnp.sum(h_ref[...].astype(jnp.float32), axis=0)
    ss, idxf = _suffix_sums(h)
    mask = ss >= numel
    b1 = jnp.max(jnp.where(mask, idxf, -1))
    ssb1 = jnp.min(jnp.where(mask, ss, jnp.float32(3e38)))
    hb1 = jnp.max(jnp.where(idxf == b1, h, -1.0))
    need2 = numel - (ssb1 - hb1)
    total = jnp.max(ss)
    b1 = jnp.maximum(b1, 0)
    b1_ref[...] = jnp.full((1, 128), b1, jnp.int32)
    need2_ref[...] = jnp.full((1, 128), need2, jnp.float32)
    tot_ref[...] = jnp.full((1, 128), total, jnp.float32)


def _scan1(h1, numel):
    return pl.pallas_call(
        functools.partial(_scan1_body, float(numel)),
        in_specs=[pl.BlockSpec((_NW, _NB1 // 128, 128), lambda: (0, 0, 0))],
        out_specs=[
            pl.BlockSpec((1, 128), lambda: (0, 0)),
            pl.BlockSpec((1, 128), lambda: (0, 0)),
            pl.BlockSpec((1, 128), lambda: (0, 0)),
        ],
        out_shape=[
            jax.ShapeDtypeStruct((1, 128), jnp.int32),
            jax.ShapeDtypeStruct((1, 128), jnp.float32),
            jax.ShapeDtypeStruct((1, 128), jnp.float32),
        ],
    )(h1.reshape(_NW, _NB1 // 128, 128))


def _tau_from_hists(numel, h_ref, b1_ref, need2_ref, tot_ref):
    h = jnp.sum(h_ref[...].astype(jnp.float32), axis=0)
    ss, idxf = _suffix_sums(h)
    need2 = need2_ref[0, 0]
    mask = ss >= need2
    b2 = jnp.max(jnp.where(mask, idxf, -1))
    b2 = jnp.maximum(b2, 0)
    b1 = b1_ref[0, 0]
    tau_bits = jnp.bitwise_or(lax.shift_left(b1, 16), lax.shift_left(b2, 1))
    tau = lax.bitcast_convert_type(tau_bits, jnp.float32)
    return jnp.where(tot_ref[0, 0] >= numel, tau, 0.0)


_BM_D = 1024
_BH_D = 1024


def _decode_body(numel, h2_ref, b1_ref, need2_ref, tot_ref, f_ref, w_ref,
                 b_ref, ftopk_ref, recon_ref, tau_s):
    m = pl.program_id(0)
    h = pl.program_id(1)

    @pl.when(jnp.logical_and(m == 0, h == 0))
    def _():
        tau_s[0] = _tau_from_hists(numel, h2_ref, b1_ref, need2_ref, tot_ref)

    tau = tau_s[0]
    f = f_ref[...]
    ft = jnp.where(f >= tau, f, 0.0)
    ftopk_ref[...] = ft
    contrib = lax.dot_general(
        ft, w_ref[...], (((1,), (1,)), ((), ())),
        preferred_element_type=jnp.float32)

    @pl.when(h == 0)
    def _():
        recon_ref[...] = contrib + b_ref[...]

    @pl.when(h != 0)
    def _():
        recon_ref[...] += contrib


def _decode(h2, b1_arr, need2_arr, tot_arr, f, W, b_dec, numel):
    B, H = f.shape
    D = W.shape[0]
    grid = (B // _BM_D, H // _BH_D)
    return pl.pallas_call(
        functools.partial(_decode_body, float(numel)),
        grid=grid,
        in_specs=[
            pl.BlockSpec((_NW, _NB2 // 128, 128), lambda m, h: (0, 0, 0)),
            pl.BlockSpec(memory_space=pltpu.SMEM),
            pl.BlockSpec(memory_space=pltpu.SMEM),
            pl.BlockSpec(memory_space=pltpu.SMEM),
            pl.BlockSpec((_BM_D, _BH_D), lambda m, h: (m, h)),
            pl.BlockSpec((D, _BH_D), lambda m, h: (0, h)),
            pl.BlockSpec((1, D), lambda m, h: (0, 0)),
        ],
        out_specs=[
            pl.BlockSpec((_BM_D, _BH_D), lambda m, h: (m, h)),
            pl.BlockSpec((_BM_D, D), lambda m, h: (m, 0)),
        ],
        out_shape=[
            jax.ShapeDtypeStruct((B, H), jnp.float32),
            jax.ShapeDtypeStruct((B, D), jnp.float32),
        ],
        scratch_shapes=[pltpu.SMEM((1,), jnp.float32)],
    )(h2.reshape(_NW, _NB2 // 128, 128), b1_arr, need2_arr, tot_arr,
      f, W, b_dec.reshape(1, D))


def kernel(x, W, b_enc, b_dec):
    B = x.shape[0]
    numel = 64 * B
    f = _encode(x, W, b_enc)
    h1 = _hist1(f)
    b1_arr, need2_arr, tot_arr = _scan1(h1, numel)
    h2 = _hist2(f, b1_arr.reshape(128))
    f_topk, recon = _decode(h2, b1_arr, need2_arr, tot_arr, f, W, b_dec,
                            numel)
    return (recon, f_topk)


# speedup vs baseline: 164.8741x; 1.0285x over previous
import functools

import jax
import jax.numpy as jnp
from jax import lax
from jax.experimental import pallas as pl
from jax.experimental.pallas import tpu as pltpu
from jax.experimental.pallas import tpu_sc as plsc

_NC = 2
_NS = 16
_NW = _NC * _NS
_LANES = 16

_NB1 = 32768
_NB2 = 32768
_CH_ROWS = 8
_CH_COLS = 2048


_BM_E = 512
_BN_E = 4096


def _encode_body(x_ref, w_ref, b_ref, f_ref):
    acc = jnp.dot(x_ref[...], w_ref[...], preferred_element_type=jnp.float32)
    f_ref[...] = jnp.maximum(acc + b_ref[...], 0.0)


def _encode(x, W, b_enc):
    B, D = x.shape
    H = W.shape[1]
    grid = (H // _BN_E, B // _BM_E)
    return pl.pallas_call(
        _encode_body,
        grid=grid,
        in_specs=[
            pl.BlockSpec((_BM_E, D), lambda n, m: (m, 0)),
            pl.BlockSpec((D, _BN_E), lambda n, m: (0, n)),
            pl.BlockSpec((1, _BN_E), lambda n, m: (0, n)),
        ],
        out_specs=pl.BlockSpec((_BM_E, _BN_E), lambda n, m: (m, n)),
        out_shape=jax.ShapeDtypeStruct((B, H), jnp.float32),
    )(x, W, b_enc.reshape(1, H))


def _zero_vmem(ref, n):
    zeros = jnp.zeros((_LANES,), jnp.int32)

    def body(i, _):
        ref[pl.ds(i * _LANES, _LANES)] = zeros
        return 0

    lax.fori_loop(0, n // _LANES, body, 0)


def _hist_pass(f_hbm, out_hbm, buf0, buf1, hist, sem0, sem1, *,
               nbins, record):
    wid = lax.axis_index("s") * _NC + lax.axis_index("c")
    nrows, ncols = f_hbm.shape
    rows_per_w = nrows // _NW
    row0 = wid * rows_per_w
    ncolc = ncols // _CH_COLS
    nch = (rows_per_w // _CH_ROWS) * ncolc
    _zero_vmem(hist, nbins)

    ones = jnp.ones((_LANES,), jnp.int32)
    nvec = _CH_COLS // _LANES

    def _slc(t):
        return (pl.ds(row0 + (t // ncolc) * _CH_ROWS, _CH_ROWS),
                pl.ds((t % ncolc) * _CH_COLS, _CH_COLS))

    def start(t, buf, sem):
        r, c = _slc(t)
        pltpu.async_copy(f_hbm.at[r, c], buf, sem)

    def wait(t, buf, sem):
        r, c = _slc(t)
        pltpu.make_async_copy(f_hbm.at[r, c], buf, sem).wait()

    def process(buf):
        for r in range(_CH_ROWS):
            @plsc.parallel_loop(0, nvec, unroll=8)
            def _(i):
                v = buf[r, pl.ds(i * _LANES, _LANES)]
                bits = lax.bitcast_convert_type(v, jnp.int32)
                record(hist, bits, ones)

    start(0, buf0, sem0)
    start(1, buf1, sem1)

    def outer(t, _):
        wait(2 * t, buf0, sem0)
        process(buf0)

        @pl.when(t < nch // 2 - 1)
        def _():
            start(2 * t + 2, buf0, sem0)

        wait(2 * t + 1, buf1, sem1)
        process(buf1)

        @pl.when(t < nch // 2 - 1)
        def _():
            start(2 * t + 3, buf1, sem1)

        return 0

    lax.fori_loop(0, nch // 2, outer, 0)
    pltpu.sync_copy(hist, out_hbm.at[wid])


def _hist1(f):
    mesh = plsc.VectorSubcoreMesh(core_axis_name="c", subcore_axis_name="s")

    def record(hist, bits, ones):
        idx = lax.shift_right_logical(bits, 16)
        plsc.addupdate_scatter(hist, [idx], ones, mask=bits > 0)

    @functools.partial(
        pl.kernel, mesh=mesh,
        compiler_params=pltpu.CompilerParams(needs_layout_passes=False),
        out_type=jax.ShapeDtypeStruct((_NW, _NB1), jnp.int32),
        scratch_types=[
            pltpu.VMEM((_CH_ROWS, _CH_COLS), jnp.float32),
            pltpu.VMEM((_CH_ROWS, _CH_COLS), jnp.float32),
            pltpu.VMEM((_NB1,), jnp.int32),
            pltpu.SemaphoreType.DMA,
            pltpu.SemaphoreType.DMA,
        ],
    )
    def k(f_hbm, out_hbm, buf0, buf1, hist, sem0, sem1):
        _hist_pass(f_hbm, out_hbm, buf0, buf1, hist, sem0, sem1,
                   nbins=_NB1, record=record)

    return k(f)


def _hist2(f, b1_arr):
    mesh = plsc.VectorSubcoreMesh(core_axis_name="c", subcore_axis_name="s")

    @functools.partial(
        pl.kernel, mesh=mesh,
        compiler_params=pltpu.CompilerParams(needs_layout_passes=False),
        out_type=jax.ShapeDtypeStruct((_NW, _NB2), jnp.int32),
        scratch_types=[
            pltpu.VMEM((_CH_ROWS, _CH_COLS), jnp.float32),
            pltpu.VMEM((_CH_ROWS, _CH_COLS), jnp.float32),
            pltpu.VMEM((_NB2,), jnp.int32),
            pltpu.VMEM((_LANES,), jnp.int32),
            pltpu.SemaphoreType.DMA,
            pltpu.SemaphoreType.DMA,
        ],
    )
    def k(f_hbm, b1_hbm, out_hbm, buf0, buf1, hist, b1_buf, sem0, sem1):
        pltpu.sync_copy(b1_hbm.at[pl.ds(0, _LANES)], b1_buf)
        b1v = b1_buf[...]

        def record(hist_ref, bits, ones):
            top = lax.shift_right_logical(bits, 16)
            q = jnp.bitwise_and(lax.shift_right_logical(bits, 1), 0x7FFF)
            mask = jnp.logical_and(top == b1v, bits > 0)
            plsc.addupdate_scatter(hist_ref, [q], ones, mask=mask)

        _hist_pass(f_hbm, out_hbm, buf0, buf1, hist, sem0, sem1,
                   nbins=_NB2, record=record)

    return k(f, b1_arr)


def _suffix_sums(h):
    R = h.shape[0]
    kk = lax.broadcasted_iota(jnp.int32, (128, 128), 0)
    cc = lax.broadcasted_iota(jnp.int32, (128, 128), 1)
    lt = (kk >= cc).astype(jnp.float32)
    ws = jnp.dot(h, lt, preferred_element_type=jnp.float32,
                 precision=lax.Precision.HIGHEST)
    rr = lax.broadcasted_iota(jnp.int32, (R, R), 0)
    rc = lax.broadcasted_iota(jnp.int32, (R, R), 1)
    ut = (rc > rr).astype(jnp.float32)
    sre = jnp.dot(ut, ws[:, 0:1], preferred_element_type=jnp.float32,
                  precision=lax.Precision.HIGHEST)
    ss = ws + sre
    idxf = (lax.broadcasted_iota(jnp.int32, (R, 128), 0) * 128
            + lax.broadcasted_iota(jnp.int32, (R, 128), 1))
    return ss, idxf


def _scan1_body(numel, h_ref, b1_ref, need2_ref, tot_ref):
    h = jnp.sum(h_ref[...].astype(jnp.float32), axis=0)
    ss, idxf = _suffix_sums(h)
    mask = ss >= numel
    b1 = jnp.max(jnp.where(mask, idxf, -1))
    ssb1 = jnp.min(jnp.where(mask, ss, jnp.float32(3e38)))
    hb1 = jnp.max(jnp.where(idxf == b1, h, -1.0))
    need2 = numel - (ssb1 - hb1)
    total = jnp.max(ss)
    b1 = jnp.maximum(b1, 0)
    b1_ref[...] = jnp.full((1, 128), b1, jnp.int32)
    need2_ref[...] = jnp.full((1, 128), need2, jnp.float32)
    tot_ref[...] = jnp.full((1, 128), total, jnp.float32)


def _scan1(h1, numel):
    return pl.pallas_call(
        functools.partial(_scan1_body, float(numel)),
        in_specs=[pl.BlockSpec((_NW, _NB1 // 128, 128), lambda: (0, 0, 0))],
        out_specs=[
            pl.BlockSpec((1, 128), lambda: (0, 0)),
            pl.BlockSpec((1, 128), lambda: (0, 0)),
            pl.BlockSpec((1, 128), lambda: (0, 0)),
        ],
        out_shape=[
            jax.ShapeDtypeStruct((1, 128), jnp.int32),
            jax.ShapeDtypeStruct((1, 128), jnp.float32),
            jax.ShapeDtypeStruct((1, 128), jnp.float32),
        ],
    )(h1.reshape(_NW, _NB1 // 128, 128))


def _tau_from_hists(numel, h_ref, b1_ref, need2_ref, tot_ref):
    h = jnp.sum(h_ref[...].astype(jnp.float32), axis=0)
    ss, idxf = _suffix_sums(h)
    need2 = need2_ref[0, 0]
    mask = ss >= need2
    b2 = jnp.max(jnp.where(mask, idxf, -1))
    b2 = jnp.maximum(b2, 0)
    b1 = b1_ref[0, 0]
    tau_bits = jnp.bitwise_or(lax.shift_left(b1, 16), lax.shift_left(b2, 1))
    tau = lax.bitcast_convert_type(tau_bits, jnp.float32)
    return jnp.where(tot_ref[0, 0] >= numel, tau, 0.0)


_BM_D = 1024
_BH_D = 1024


def _decode_body(numel, h2_ref, b1_ref, need2_ref, tot_ref, f_ref, w_ref,
                 b_ref, ftopk_ref, recon_ref, tau_s):
    m = pl.program_id(0)
    h = pl.program_id(1)

    @pl.when(jnp.logical_and(m == 0, h == 0))
    def _():
        tau_s[0] = _tau_from_hists(numel, h2_ref, b1_ref, need2_ref, tot_ref)

    tau = tau_s[0]
    f = f_ref[...]
    ft = jnp.where(f >= tau, f, 0.0)
    ftopk_ref[...] = ft
    contrib = lax.dot_general(
        ft, w_ref[...], (((1,), (1,)), ((), ())),
        preferred_element_type=jnp.float32)

    @pl.when(h == 0)
    def _():
        recon_ref[...] = contrib + b_ref[...]

    @pl.when(h != 0)
    def _():
        recon_ref[...] += contrib


def _decode(h2, b1_arr, need2_arr, tot_arr, f, W, b_dec, numel):
    B, H = f.shape
    D = W.shape[0]
    grid = (B // _BM_D, H // _BH_D)
    return pl.pallas_call(
        functools.partial(_decode_body, float(numel)),
        grid=grid,
        in_specs=[
            pl.BlockSpec((_NW, _NB2 // 128, 128), lambda m, h: (0, 0, 0)),
            pl.BlockSpec(memory_space=pltpu.SMEM),
            pl.BlockSpec(memory_space=pltpu.SMEM),
            pl.BlockSpec(memory_space=pltpu.SMEM),
            pl.BlockSpec((_BM_D, _BH_D), lambda m, h: (m, h)),
            pl.BlockSpec((D, _BH_D), lambda m, h: (0, h)),
            pl.BlockSpec((1, D), lambda m, h: (0, 0)),
        ],
        out_specs=[
            pl.BlockSpec((_BM_D, _BH_D), lambda m, h: (m, h)),
            pl.BlockSpec((_BM_D, D), lambda m, h: (m, 0)),
        ],
        out_shape=[
            jax.ShapeDtypeStruct((B, H), jnp.float32),
            jax.ShapeDtypeStruct((B, D), jnp.float32),
        ],
        scratch_shapes=[pltpu.SMEM((1,), jnp.float32)],
    )(h2.reshape(_NW, _NB2 // 128, 128), b1_arr, need2_arr, tot_arr,
      f, W, b_dec.reshape(1, D))


def kernel(x, W, b_enc, b_dec):
    B = x.shape[0]
    numel = 64 * B
    f = _encode(x, W, b_enc)
    h1 = _hist1(f)
    b1_arr, need2_arr, tot_arr = _scan1(h1, numel)
    h2 = _hist2(f, b1_arr.reshape(128))
    f_topk, recon = _decode(h2, b1_arr, need2_arr, tot_arr, f, W, b_dec,
                            numel)
    return (recon, f_topk)
